# baseline (speedup 1.0000x reference)
import functools

import jax
import jax.numpy as jnp
from jax import lax
from jax.experimental import pallas as pl
from jax.experimental.pallas import tpu as pltpu
from jax.experimental.pallas import tpu_sc as plsc

VOCAB = 1000000
D = 64
B = 16384
W = 20
L = 16
NC = 2
NS = 16
NW = NC * NS
CB = B // NW
NB = 32
NCHUNK = CB // NB
PAIRS = NB * W
IDXROWS = PAIRS // 128

_GATHER_DNUMS = lax.GatherDimensionNumbers(
    offset_dims=(), collapsed_slice_dims=(0,), start_index_map=(0,))


def _shuffle(x, perm):
    return lax.gather(x, perm[:, None], dimension_numbers=_GATHER_DNUMS,
                      slice_sizes=(1,),
                      mode=lax.GatherScatterMode.PROMISE_IN_BOUNDS)


def _sc_body(cidx_hbm, tidx_hbm, nidx_hbm, vtab_hbm, utab_hbm,
             pos_hbm, neg_hbm,
             cidx_v, tidx_v, nidx_v, vrows_v, trows_v, nrows_v,
             pos_v, neg_v, sem):
    c = lax.axis_index("c")
    s = lax.axis_index("s")
    wid = s * NC + c
    iota = lax.iota(jnp.int32, L)

    def chunk_body(ck, _):
        cbase = wid * CB + ck * NB
        pbase = cbase * W

        pltpu.sync_copy(cidx_hbm.at[pl.ds(cbase, NB)], cidx_v)
        pltpu.sync_copy(tidx_hbm.at[pl.ds(pbase, PAIRS)], tidx_v)
        pltpu.sync_copy(nidx_hbm.at[pl.ds(pbase, PAIRS)], nidx_v)

        descs = [pltpu.async_copy(vtab_hbm.at[cidx_v], vrows_v, sem)]
        for j in range(IDXROWS):
            descs.append(pltpu.async_copy(
                utab_hbm.at[tidx_v.at[pl.ds(j * 128, 128)]],
                trows_v.at[pl.ds(j * 128, 128)], sem))
            descs.append(pltpu.async_copy(
                utab_hbm.at[nidx_v.at[pl.ds(j * 128, 128)]],
                nrows_v.at[pl.ds(j * 128, 128)], sem))
        for dsc in descs:
            dsc.wait()

        def center_body(i, carry):
            vs = [vrows_v[i, pl.ds(k * L, L)] for k in range(D // L)]

            def pair_body(w, c2):
                pv, nv = c2
                p = i * W + w
                pacc = trows_v[p, pl.ds(0, L)] * vs[0]
                nacc = nrows_v[p, pl.ds(0, L)] * vs[0]
                for k in range(1, D // L):
                    pacc = pacc + trows_v[p, pl.ds(k * L, L)] * vs[k]
                    nacc = nacc + nrows_v[p, pl.ds(k * L, L)] * vs[k]
                for sh in (8, 4, 2, 1):
                    perm = iota ^ sh
                    pacc = pacc + _shuffle(pacc, perm)
                    nacc = nacc + _shuffle(nacc, perm)
                lane = lax.rem(p, L)
                pv = jnp.where(iota == lane, pacc, pv)
                nv = jnp.where(iota == lane, nacc, nv)

                @pl.when(lane == L - 1)
                def _():
                    gbase = pl.multiple_of(p - (L - 1), L)
                    pos_v[pl.ds(gbase, L)] = pv
                    neg_v[pl.ds(gbase, L)] = nv

                return (pv, nv)

            return lax.fori_loop(0, W, pair_body, carry)

        zeros = jnp.zeros((L,), jnp.float32)
        lax.fori_loop(0, NB, center_body, (zeros, zeros))

        pltpu.sync_copy(pos_v, pos_hbm.at[pl.ds(pbase, PAIRS)])
        pltpu.sync_copy(neg_v, neg_hbm.at[pl.ds(pbase, PAIRS)])
        return 0

    lax.fori_loop(0, NCHUNK, chunk_body, 0)


_sc_scores = functools.partial(
    pl.kernel,
    out_type=(jax.ShapeDtypeStruct((B * W,), jnp.float32),
              jax.ShapeDtypeStruct((B * W,), jnp.float32)),
    mesh=plsc.VectorSubcoreMesh(core_axis_name="c", subcore_axis_name="s"),
    compiler_params=pltpu.CompilerParams(use_tc_tiling_on_sc=False),
    scratch_types=[
        pltpu.VMEM((NB,), jnp.int32),
        pltpu.VMEM((PAIRS,), jnp.int32),
        pltpu.VMEM((PAIRS,), jnp.int32),
        pltpu.VMEM((NB, D), jnp.float32),
        pltpu.VMEM((PAIRS, D), jnp.float32),
        pltpu.VMEM((PAIRS, D), jnp.float32),
        pltpu.VMEM((PAIRS,), jnp.float32),
        pltpu.VMEM((PAIRS,), jnp.float32),
        pltpu.SemaphoreType.DMA,
    ],
)(_sc_body)


def _loss_body(pos_ref, neg_ref, out_ref):
    p = pos_ref[...]
    n = neg_ref[...]
    sp_p = jnp.maximum(-p, 0.0) + jnp.log1p(jnp.exp(-jnp.abs(p)))
    sp_n = jnp.maximum(n, 0.0) + jnp.log1p(jnp.exp(-jnp.abs(n)))
    out_ref[...] = sp_p + sp_n


_ROWS = B * W // 128


def kernel(center_word, target_word, negative_word, emb_v_table, emb_u_table):
    cidx = center_word.reshape(B).astype(jnp.int32)
    tidx = target_word.reshape(B * W).astype(jnp.int32)
    nidx = negative_word.reshape(B * W).astype(jnp.int32)
    pos, neg = _sc_scores(cidx, tidx, nidx, emb_v_table, emb_u_table)
    loss = pl.pallas_call(
        _loss_body,
        out_shape=jax.ShapeDtypeStruct((_ROWS, 128), jnp.float32),
    )(pos.reshape(_ROWS, 128), neg.reshape(_ROWS, 128))
    return loss.reshape(B, W)
